# Initial kernel scaffold; baseline (speedup 1.0000x reference)
import jax
import jax.numpy as jnp
from jax.experimental import pallas as pl


def kernel(x, edge_index, W1, b1, W2, b2):
    raise NotImplementedError("write your pallas kernel here")


# speedup vs baseline: 24.7074x; 24.7074x over previous
import functools

import jax
import jax.numpy as jnp
from jax import lax
from jax.experimental import pallas as pl
from jax.experimental.pallas import tpu as pltpu
from jax.experimental.pallas import tpu_sc as plsc

N = 10000
E = 320000
IN_DIM = 128
HID_DIM = 32
OUT_DIM = 16

NPAD = 10240
NC = 2
NS = 16
NW = NC * NS
CH = 128
NCH = 80
EPW = NCH * CH
EPAD = NW * EPW
RPT = NPAD // NS
DEGW = 8

@functools.lru_cache(maxsize=None)
def _mesh():
    return plsc.VectorSubcoreMesh(core_axis_name="c", subcore_axis_name="s")


def _sc_degree(dst_hbm, ones_hbm, zeros_hbm, out_hbm, acc_sh, dst_v, ones_v, sem):
    cid = lax.axis_index("c")
    sid = lax.axis_index("s")
    wid = cid * NS + sid
    r0 = sid * RPT
    pltpu.sync_copy(zeros_hbm, acc_sh.at[pl.ds(r0, RPT)])
    pltpu.sync_copy(dst_hbm.at[wid], dst_v)
    pltpu.sync_copy(ones_hbm, ones_v)
    plsc.subcore_barrier()

    def step(j, carry):
        pltpu.sync_copy(ones_v, acc_sh.at[dst_v.at[j]], add=True)
        return carry

    lax.fori_loop(0, NCH, step, 0)
    plsc.subcore_barrier()
    pltpu.sync_copy(acc_sh.at[pl.ds(r0, RPT)], out_hbm.at[cid, pl.ds(r0, RPT)])


@functools.lru_cache(maxsize=None)
def _degree_call():
    return functools.partial(
        pl.kernel,
        out_type=jax.ShapeDtypeStruct((NC, NPAD, DEGW), jnp.float32),
        mesh=_mesh(),
        compiler_params=pltpu.CompilerParams(use_tc_tiling_on_sc=False),
        scratch_types=[
            pltpu.VMEM_SHARED((NPAD, DEGW), jnp.float32),
            pltpu.VMEM((NCH, CH), jnp.int32),
            pltpu.VMEM((CH, DEGW), jnp.float32),
            pltpu.SemaphoreType.DMA,
        ],
    )(_sc_degree)


def _make_sc_agg(D):
    def _sc_agg(src_hbm, dst_hbm, m_hbm, zeros_hbm, out_hbm,
                acc_sh, src_v, dst_v, buf, sem):
        cid = lax.axis_index("c")
        sid = lax.axis_index("s")
        wid = cid * NS + sid
        r0 = sid * RPT
        pltpu.sync_copy(zeros_hbm, acc_sh.at[pl.ds(r0, RPT)])
        pltpu.sync_copy(src_hbm.at[wid], src_v)
        pltpu.sync_copy(dst_hbm.at[wid], dst_v)
        plsc.subcore_barrier()

        def step(j, carry):
            pltpu.async_copy(m_hbm.at[src_v.at[j]], buf, sem).wait()
            pltpu.sync_copy(buf, acc_sh.at[dst_v.at[j]], add=True)
            return carry

        lax.fori_loop(0, NCH, step, 0)
        plsc.subcore_barrier()
        pltpu.sync_copy(acc_sh.at[pl.ds(r0, RPT)],
                        out_hbm.at[cid, pl.ds(r0, RPT)])

    return functools.partial(
        pl.kernel,
        out_type=jax.ShapeDtypeStruct((NC, NPAD, D), jnp.float32),
        mesh=_mesh(),
        compiler_params=pltpu.CompilerParams(use_tc_tiling_on_sc=False),
        scratch_types=[
            pltpu.VMEM_SHARED((NPAD, D), jnp.float32),
            pltpu.VMEM((NCH, CH), jnp.int32),
            pltpu.VMEM((NCH, CH), jnp.int32),
            pltpu.VMEM((CH, D), jnp.float32),
            pltpu.SemaphoreType.DMA,
        ],
    )(_sc_agg)


_make_sc_agg = functools.lru_cache(maxsize=None)(_make_sc_agg)


_BLK = 1024
_GRID = NPAD // _BLK


def _tc_a(x_ref, w1_ref, degp_ref, m1_ref, dinv_ref):
    deg = degp_ref[0, :, 0] + degp_ref[1, :, 0] + 1.0
    dinv = lax.rsqrt(deg).reshape(_BLK, 1)
    h = jnp.dot(x_ref[...], w1_ref[...], preferred_element_type=jnp.float32)
    m1_ref[...] = dinv * h
    dinv_ref[...] = dinv


def _tc_b(m1_ref, acc_ref, dinv_ref, b1_ref, w2_ref, m2_ref):
    dinv = dinv_ref[...]
    s = m1_ref[...] + acc_ref[0] + acc_ref[1]
    z = jnp.maximum(dinv * s + b1_ref[...], 0.0)
    m2_ref[...] = dinv * jnp.dot(z, w2_ref[...], preferred_element_type=jnp.float32)


def _tc_c(m2_ref, acc_ref, dinv_ref, b2_ref, out_ref):
    dinv = dinv_ref[...]
    s = m2_ref[...] + acc_ref[0] + acc_ref[1]
    out_ref[...] = dinv * s + b2_ref[...]


def _tc_a_call(x_pad, W1, degp):
    return pl.pallas_call(
        _tc_a,
        grid=(_GRID,),
        in_specs=[
            pl.BlockSpec((_BLK, IN_DIM), lambda i: (i, 0)),
            pl.BlockSpec((IN_DIM, HID_DIM), lambda i: (0, 0)),
            pl.BlockSpec((NC, _BLK, DEGW), lambda i: (0, i, 0)),
        ],
        out_specs=[
            pl.BlockSpec((_BLK, HID_DIM), lambda i: (i, 0)),
            pl.BlockSpec((_BLK, 1), lambda i: (i, 0)),
        ],
        out_shape=[
            jax.ShapeDtypeStruct((NPAD, HID_DIM), jnp.float32),
            jax.ShapeDtypeStruct((NPAD, 1), jnp.float32),
        ],
    )(x_pad, W1, degp)


def _tc_b_call(m1, acc1, dinv, b1, W2):
    return pl.pallas_call(
        _tc_b,
        grid=(_GRID,),
        in_specs=[
            pl.BlockSpec((_BLK, HID_DIM), lambda i: (i, 0)),
            pl.BlockSpec((NC, _BLK, HID_DIM), lambda i: (0, i, 0)),
            pl.BlockSpec((_BLK, 1), lambda i: (i, 0)),
            pl.BlockSpec((1, HID_DIM), lambda i: (0, 0)),
            pl.BlockSpec((HID_DIM, OUT_DIM), lambda i: (0, 0)),
        ],
        out_specs=pl.BlockSpec((_BLK, OUT_DIM), lambda i: (i, 0)),
        out_shape=jax.ShapeDtypeStruct((NPAD, OUT_DIM), jnp.float32),
    )(m1, acc1, dinv, b1, W2)


def _tc_c_call(m2, acc2, dinv, b2):
    return pl.pallas_call(
        _tc_c,
        grid=(_GRID,),
        in_specs=[
            pl.BlockSpec((_BLK, OUT_DIM), lambda i: (i, 0)),
            pl.BlockSpec((NC, _BLK, OUT_DIM), lambda i: (0, i, 0)),
            pl.BlockSpec((_BLK, 1), lambda i: (i, 0)),
            pl.BlockSpec((1, OUT_DIM), lambda i: (0, 0)),
        ],
        out_specs=pl.BlockSpec((_BLK, OUT_DIM), lambda i: (i, 0)),
        out_shape=jax.ShapeDtypeStruct((NPAD, OUT_DIM), jnp.float32),
    )(m2, acc2, dinv, b2)


def kernel(x, edge_index, W1, b1, W2, b2):
    src = jnp.pad(edge_index[0], (0, EPAD - E), constant_values=N).reshape(NW, NCH, CH)
    dst = jnp.pad(edge_index[1], (0, EPAD - E), constant_values=N).reshape(NW, NCH, CH)
    x_pad = jnp.pad(x, ((0, NPAD - N), (0, 0)))

    ones_deg = jnp.ones((CH, DEGW), jnp.float32)
    zeros_deg = jnp.zeros((RPT, DEGW), jnp.float32)
    zeros32 = jnp.zeros((RPT, HID_DIM), jnp.float32)
    zeros16 = jnp.zeros((RPT, OUT_DIM), jnp.float32)

    degp = _degree_call()(dst, ones_deg, zeros_deg)
    m1, dinv = _tc_a_call(x_pad, W1, degp)
    acc1 = _make_sc_agg(HID_DIM)(src, dst, m1, zeros32)
    m2 = _tc_b_call(m1, acc1, dinv, b1.reshape(1, HID_DIM), W2)
    acc2 = _make_sc_agg(OUT_DIM)(src, dst, m2, zeros16)
    out = _tc_c_call(m2, acc2, dinv, b2.reshape(1, OUT_DIM))
    return out[:N]


# speedup vs baseline: 42.4490x; 1.7181x over previous
import functools

import jax
import jax.numpy as jnp
from jax import lax
from jax.experimental import pallas as pl
from jax.experimental.pallas import tpu as pltpu
from jax.experimental.pallas import tpu_sc as plsc

N = 10000
E = 320000
IN_DIM = 128
HID_DIM = 32
OUT_DIM = 16

NPAD = 10240
NC = 2
NS = 16
NW = NC * NS
CH = 125
NCH = 80
EPW = NCH * CH
RPT = NPAD // NS
DEGW = 8


@functools.lru_cache(maxsize=None)
def _mesh():
    return plsc.VectorSubcoreMesh(core_axis_name="c", subcore_axis_name="s")


def _sc_degree(dst_hbm, ones_hbm, zeros_hbm, out_hbm, acc_sh, dst_v, ones_v, sem):
    cid = lax.axis_index("c")
    sid = lax.axis_index("s")
    wid = cid * NS + sid
    r0 = sid * RPT
    pltpu.sync_copy(zeros_hbm, acc_sh.at[pl.ds(r0, RPT)])
    pltpu.sync_copy(dst_hbm.at[wid], dst_v)
    pltpu.sync_copy(ones_hbm, ones_v)
    plsc.subcore_barrier()

    def step(j, carry):
        pltpu.sync_copy(ones_v, acc_sh.at[dst_v.at[j]], add=True)
        return carry

    lax.fori_loop(0, NCH, step, 0)
    plsc.subcore_barrier()
    pltpu.sync_copy(acc_sh.at[pl.ds(r0, RPT)], out_hbm.at[cid, pl.ds(r0, RPT)])


@functools.lru_cache(maxsize=None)
def _degree_call():
    return functools.partial(
        pl.kernel,
        out_type=jax.ShapeDtypeStruct((NC, NPAD, DEGW), jnp.float32),
        mesh=_mesh(),
        compiler_params=pltpu.CompilerParams(use_tc_tiling_on_sc=False),
        scratch_types=[
            pltpu.VMEM_SHARED((NPAD, DEGW), jnp.float32),
            pltpu.VMEM((NCH, CH), jnp.int32),
            pltpu.VMEM((CH, DEGW), jnp.float32),
            pltpu.SemaphoreType.DMA,
        ],
    )(_sc_degree)


def _make_sc_agg(D):
    def _sc_agg(src_hbm, dst_hbm, m_hbm, zeros_hbm, out_hbm,
                acc_sh, src_v, dst_v, buf_a, buf_b, sem_a, sem_b):
        cid = lax.axis_index("c")
        sid = lax.axis_index("s")
        wid = cid * NS + sid
        r0 = sid * RPT
        pltpu.sync_copy(zeros_hbm, acc_sh.at[pl.ds(r0, RPT)])
        pltpu.sync_copy(src_hbm.at[wid], src_v)
        pltpu.sync_copy(dst_hbm.at[wid], dst_v)
        plsc.subcore_barrier()

        pltpu.async_copy(m_hbm.at[src_v.at[0]], buf_a, sem_a)
        pltpu.async_copy(m_hbm.at[src_v.at[1]], buf_b, sem_b)

        def step(t, carry):
            j0 = 2 * t
            j1 = 2 * t + 1
            pltpu.make_async_copy(m_hbm.at[src_v.at[j0]], buf_a, sem_a).wait()
            pltpu.sync_copy(buf_a, acc_sh.at[dst_v.at[j0]], add=True)
            nxt_a = jnp.minimum(j0 + 2, NCH - 1)
            pltpu.async_copy(m_hbm.at[src_v.at[nxt_a]], buf_a, sem_a)
            pltpu.make_async_copy(m_hbm.at[src_v.at[j1]], buf_b, sem_b).wait()
            pltpu.sync_copy(buf_b, acc_sh.at[dst_v.at[j1]], add=True)
            nxt_b = jnp.minimum(j1 + 2, NCH - 1)
            pltpu.async_copy(m_hbm.at[src_v.at[nxt_b]], buf_b, sem_b)
            return carry

        lax.fori_loop(0, NCH // 2, step, 0)
        pltpu.make_async_copy(m_hbm.at[src_v.at[0]], buf_a, sem_a).wait()
        pltpu.make_async_copy(m_hbm.at[src_v.at[0]], buf_b, sem_b).wait()
        plsc.subcore_barrier()
        pltpu.sync_copy(acc_sh.at[pl.ds(r0, RPT)],
                        out_hbm.at[cid, pl.ds(r0, RPT)])

    return functools.partial(
        pl.kernel,
        out_type=jax.ShapeDtypeStruct((NC, NPAD, D), jnp.float32),
        mesh=_mesh(),
        compiler_params=pltpu.CompilerParams(use_tc_tiling_on_sc=False),
        scratch_types=[
            pltpu.VMEM_SHARED((NPAD, D), jnp.float32),
            pltpu.VMEM((NCH, CH), jnp.int32),
            pltpu.VMEM((NCH, CH), jnp.int32),
            pltpu.VMEM((CH, D), jnp.float32),
            pltpu.VMEM((CH, D), jnp.float32),
            pltpu.SemaphoreType.DMA,
            pltpu.SemaphoreType.DMA,
        ],
    )(_sc_agg)


_make_sc_agg = functools.lru_cache(maxsize=None)(_make_sc_agg)


_BLK = 1024
_GRID = NPAD // _BLK


def _tc_a(x_ref, w1_ref, degp_ref, m1_ref, dinv_ref):
    deg = degp_ref[0, :, 0] + degp_ref[1, :, 0] + 1.0
    dinv = lax.rsqrt(deg).reshape(_BLK, 1)
    h = jnp.dot(x_ref[...], w1_ref[...], preferred_element_type=jnp.float32)
    m1_ref[...] = dinv * h
    dinv_ref[...] = dinv


def _tc_b(m1_ref, acc_ref, dinv_ref, b1_ref, w2_ref, m2_ref):
    dinv = dinv_ref[...]
    s = m1_ref[...] + acc_ref[0] + acc_ref[1]
    z = jnp.maximum(dinv * s + b1_ref[...], 0.0)
    m2_ref[...] = dinv * jnp.dot(z, w2_ref[...], preferred_element_type=jnp.float32)


def _tc_c(m2_ref, acc_ref, dinv_ref, b2_ref, out_ref):
    dinv = dinv_ref[...]
    s = m2_ref[...] + acc_ref[0] + acc_ref[1]
    out_ref[...] = dinv * s + b2_ref[...]


def _tc_a_call(x_pad, W1, degp):
    return pl.pallas_call(
        _tc_a,
        grid=(_GRID,),
        in_specs=[
            pl.BlockSpec((_BLK, IN_DIM), lambda i: (i, 0)),
            pl.BlockSpec((IN_DIM, HID_DIM), lambda i: (0, 0)),
            pl.BlockSpec((NC, _BLK, DEGW), lambda i: (0, i, 0)),
        ],
        out_specs=[
            pl.BlockSpec((_BLK, HID_DIM), lambda i: (i, 0)),
            pl.BlockSpec((_BLK, 1), lambda i: (i, 0)),
        ],
        out_shape=[
            jax.ShapeDtypeStruct((NPAD, HID_DIM), jnp.float32),
            jax.ShapeDtypeStruct((NPAD, 1), jnp.float32),
        ],
    )(x_pad, W1, degp)


def _tc_b_call(m1, acc1, dinv, b1, W2):
    return pl.pallas_call(
        _tc_b,
        grid=(_GRID,),
        in_specs=[
            pl.BlockSpec((_BLK, HID_DIM), lambda i: (i, 0)),
            pl.BlockSpec((NC, _BLK, HID_DIM), lambda i: (0, i, 0)),
            pl.BlockSpec((_BLK, 1), lambda i: (i, 0)),
            pl.BlockSpec((1, HID_DIM), lambda i: (0, 0)),
            pl.BlockSpec((HID_DIM, OUT_DIM), lambda i: (0, 0)),
        ],
        out_specs=pl.BlockSpec((_BLK, OUT_DIM), lambda i: (i, 0)),
        out_shape=jax.ShapeDtypeStruct((NPAD, OUT_DIM), jnp.float32),
    )(m1, acc1, dinv, b1, W2)


def _tc_c_call(m2, acc2, dinv, b2):
    return pl.pallas_call(
        _tc_c,
        grid=(_GRID,),
        in_specs=[
            pl.BlockSpec((_BLK, OUT_DIM), lambda i: (i, 0)),
            pl.BlockSpec((NC, _BLK, OUT_DIM), lambda i: (0, i, 0)),
            pl.BlockSpec((_BLK, 1), lambda i: (i, 0)),
            pl.BlockSpec((1, OUT_DIM), lambda i: (0, 0)),
        ],
        out_specs=pl.BlockSpec((_BLK, OUT_DIM), lambda i: (i, 0)),
        out_shape=jax.ShapeDtypeStruct((NPAD, OUT_DIM), jnp.float32),
    )(m2, acc2, dinv, b2)


def kernel(x, edge_index, W1, b1, W2, b2):
    src = edge_index[0].reshape(NW, NCH, CH)
    dst = edge_index[1].reshape(NW, NCH, CH)
    x_pad = jnp.pad(x, ((0, NPAD - N), (0, 0)))

    ones_deg = jnp.ones((CH, DEGW), jnp.float32)
    zeros_deg = jnp.zeros((RPT, DEGW), jnp.float32)
    zeros32 = jnp.zeros((RPT, HID_DIM), jnp.float32)
    zeros16 = jnp.zeros((RPT, OUT_DIM), jnp.float32)

    degp = _degree_call()(dst, ones_deg, zeros_deg)
    m1, dinv = _tc_a_call(x_pad, W1, degp)
    acc1 = _make_sc_agg(HID_DIM)(src, dst, m1, zeros32)
    m2 = _tc_b_call(m1, acc1, dinv, b1.reshape(1, HID_DIM), W2)
    acc2 = _make_sc_agg(OUT_DIM)(src, dst, m2, zeros16)
    out = _tc_c_call(m2, acc2, dinv, b2.reshape(1, OUT_DIM))
    return out[:N]


# speedup vs baseline: 51.3842x; 1.2105x over previous
import functools

import jax
import jax.numpy as jnp
from jax import lax
from jax.experimental import pallas as pl
from jax.experimental.pallas import tpu as pltpu
from jax.experimental.pallas import tpu_sc as plsc

N = 10000
E = 320000
IN_DIM = 128
HID_DIM = 32
OUT_DIM = 16

NPAD = 10240
NC = 2
NS = 16
NW = NC * NS
CH = 125
NCH = 80
EPW = NCH * CH
RPT = NPAD // NS


@functools.lru_cache(maxsize=None)
def _mesh():
    return plsc.VectorSubcoreMesh(core_axis_name="c", subcore_axis_name="s")


def _sc_degree(dst_hbm, ones_hbm, zeros_hbm, out_hbm, acc_sh, dst_v, ones_v, sem):
    cid = lax.axis_index("c")
    sid = lax.axis_index("s")
    wid = cid * NS + sid
    r0 = sid * RPT
    pltpu.sync_copy(zeros_hbm, acc_sh.at[pl.ds(r0, RPT)])
    pltpu.sync_copy(dst_hbm.at[wid], dst_v)
    pltpu.sync_copy(ones_hbm, ones_v)
    plsc.subcore_barrier()

    def step(j, carry):
        pltpu.sync_copy(ones_v, acc_sh.at[dst_v.at[j]], add=True)
        return carry

    lax.fori_loop(0, NCH, step, 0)
    plsc.subcore_barrier()
    pltpu.sync_copy(acc_sh.at[pl.ds(r0, RPT)], out_hbm.at[cid, pl.ds(r0, RPT)])


@functools.lru_cache(maxsize=None)
def _degree_call():
    return functools.partial(
        pl.kernel,
        out_type=jax.ShapeDtypeStruct((NC, NPAD), jnp.float32),
        mesh=_mesh(),
        compiler_params=pltpu.CompilerParams(use_tc_tiling_on_sc=False),
        scratch_types=[
            pltpu.VMEM_SHARED((NPAD,), jnp.float32),
            pltpu.VMEM((NCH, CH), jnp.int32),
            pltpu.VMEM((CH,), jnp.float32),
            pltpu.SemaphoreType.DMA,
        ],
    )(_sc_degree)


def _make_sc_agg(D):
    def _sc_agg(src_hbm, dst_hbm, m_hbm, zeros_hbm, out_hbm,
                tab_sh, acc_sh, src_v, dst_v, buf_a, buf_b, sem_a, sem_b):
        cid = lax.axis_index("c")
        sid = lax.axis_index("s")
        wid = cid * NS + sid
        r0 = sid * RPT
        pltpu.sync_copy(m_hbm.at[pl.ds(r0, RPT)], tab_sh.at[pl.ds(r0, RPT)])
        pltpu.sync_copy(zeros_hbm, acc_sh.at[pl.ds(r0, RPT)])
        pltpu.sync_copy(src_hbm.at[wid], src_v)
        pltpu.sync_copy(dst_hbm.at[wid], dst_v)
        plsc.subcore_barrier()

        pltpu.async_copy(tab_sh.at[src_v.at[0]], buf_a, sem_a)
        pltpu.async_copy(tab_sh.at[src_v.at[1]], buf_b, sem_b)

        def step(t, carry):
            j0 = 2 * t
            j1 = 2 * t + 1
            pltpu.make_async_copy(tab_sh.at[src_v.at[j0]], buf_a, sem_a).wait()
            pltpu.sync_copy(buf_a, acc_sh.at[dst_v.at[j0]], add=True)
            nxt_a = jnp.minimum(j0 + 2, NCH - 1)
            pltpu.async_copy(tab_sh.at[src_v.at[nxt_a]], buf_a, sem_a)
            pltpu.make_async_copy(tab_sh.at[src_v.at[j1]], buf_b, sem_b).wait()
            pltpu.sync_copy(buf_b, acc_sh.at[dst_v.at[j1]], add=True)
            nxt_b = jnp.minimum(j1 + 2, NCH - 1)
            pltpu.async_copy(tab_sh.at[src_v.at[nxt_b]], buf_b, sem_b)
            return carry

        lax.fori_loop(0, NCH // 2, step, 0)
        pltpu.make_async_copy(tab_sh.at[src_v.at[0]], buf_a, sem_a).wait()
        pltpu.make_async_copy(tab_sh.at[src_v.at[0]], buf_b, sem_b).wait()
        plsc.subcore_barrier()
        pltpu.sync_copy(acc_sh.at[pl.ds(r0, RPT)],
                        out_hbm.at[cid, pl.ds(r0, RPT)])

    return functools.partial(
        pl.kernel,
        out_type=jax.ShapeDtypeStruct((NC, NPAD, D), jnp.float32),
        mesh=_mesh(),
        compiler_params=pltpu.CompilerParams(use_tc_tiling_on_sc=False),
        scratch_types=[
            pltpu.VMEM_SHARED((NPAD, D), jnp.float32),
            pltpu.VMEM_SHARED((NPAD, D), jnp.float32),
            pltpu.VMEM((NCH, CH), jnp.int32),
            pltpu.VMEM((NCH, CH), jnp.int32),
            pltpu.VMEM((CH, D), jnp.float32),
            pltpu.VMEM((CH, D), jnp.float32),
            pltpu.SemaphoreType.DMA,
            pltpu.SemaphoreType.DMA,
        ],
    )(_sc_agg)


_make_sc_agg = functools.lru_cache(maxsize=None)(_make_sc_agg)


_BLK = 1024
_GRID = NPAD // _BLK
_OBLK = 1000
_OGRID = N // _OBLK


def _tc_m(x_ref, w1_ref, h1_ref):
    h1_ref[...] = jnp.dot(x_ref[...], w1_ref[...],
                          preferred_element_type=jnp.float32)


def _tc_a(h1_ref, degp_ref, m1_ref, dinv_ref):
    deg = degp_ref[0] + degp_ref[1] + 1.0
    dinv = lax.rsqrt(deg).reshape(_BLK, 1)
    m1_ref[...] = dinv * h1_ref[...]
    dinv_ref[...] = dinv


def _tc_b(m1_ref, acc_ref, dinv_ref, b1_ref, w2_ref, m2_ref):
    dinv = dinv_ref[...]
    s = m1_ref[...] + acc_ref[0] + acc_ref[1]
    z = jnp.maximum(dinv * s + b1_ref[...], 0.0)
    m2_ref[...] = dinv * jnp.dot(z, w2_ref[...], preferred_element_type=jnp.float32)


def _tc_c(m2_ref, acc_ref, dinv_ref, b2_ref, out_ref):
    dinv = dinv_ref[...]
    s = m2_ref[...] + acc_ref[0] + acc_ref[1]
    out_ref[...] = dinv * s + b2_ref[...]


def _tc_m_call(x_pad, W1):
    return pl.pallas_call(
        _tc_m,
        grid=(_GRID,),
        in_specs=[
            pl.BlockSpec((_BLK, IN_DIM), lambda i: (i, 0)),
            pl.BlockSpec((IN_DIM, HID_DIM), lambda i: (0, 0)),
        ],
        out_specs=pl.BlockSpec((_BLK, HID_DIM), lambda i: (i, 0)),
        out_shape=jax.ShapeDtypeStruct((NPAD, HID_DIM), jnp.float32),
    )(x_pad, W1)


def _tc_a_call(h1, degp):
    return pl.pallas_call(
        _tc_a,
        grid=(_GRID,),
        in_specs=[
            pl.BlockSpec((_BLK, HID_DIM), lambda i: (i, 0)),
            pl.BlockSpec((NC, _BLK), lambda i: (0, i)),
        ],
        out_specs=[
            pl.BlockSpec((_BLK, HID_DIM), lambda i: (i, 0)),
            pl.BlockSpec((_BLK, 1), lambda i: (i, 0)),
        ],
        out_shape=[
            jax.ShapeDtypeStruct((NPAD, HID_DIM), jnp.float32),
            jax.ShapeDtypeStruct((NPAD, 1), jnp.float32),
        ],
    )(h1, degp)


def _tc_b_call(m1, acc1, dinv, b1, W2):
    return pl.pallas_call(
        _tc_b,
        grid=(_GRID,),
        in_specs=[
            pl.BlockSpec((_BLK, HID_DIM), lambda i: (i, 0)),
            pl.BlockSpec((NC, _BLK, HID_DIM), lambda i: (0, i, 0)),
            pl.BlockSpec((_BLK, 1), lambda i: (i, 0)),
            pl.BlockSpec((1, HID_DIM), lambda i: (0, 0)),
            pl.BlockSpec((HID_DIM, OUT_DIM), lambda i: (0, 0)),
        ],
        out_specs=pl.BlockSpec((_BLK, OUT_DIM), lambda i: (i, 0)),
        out_shape=jax.ShapeDtypeStruct((NPAD, OUT_DIM), jnp.float32),
    )(m1, acc1, dinv, b1, W2)


def _tc_c_call(m2, acc2, dinv, b2):
    return pl.pallas_call(
        _tc_c,
        grid=(_OGRID,),
        in_specs=[
            pl.BlockSpec((_OBLK, OUT_DIM), lambda i: (i, 0)),
            pl.BlockSpec((NC, _OBLK, OUT_DIM), lambda i: (0, i, 0)),
            pl.BlockSpec((_OBLK, 1), lambda i: (i, 0)),
            pl.BlockSpec((1, OUT_DIM), lambda i: (0, 0)),
        ],
        out_specs=pl.BlockSpec((_OBLK, OUT_DIM), lambda i: (i, 0)),
        out_shape=jax.ShapeDtypeStruct((N, OUT_DIM), jnp.float32),
    )(m2, acc2, dinv, b2)


def kernel(x, edge_index, W1, b1, W2, b2):
    src = edge_index[0].reshape(NW, NCH, CH)
    dst = edge_index[1].reshape(NW, NCH, CH)
    x_pad = jnp.pad(x, ((0, NPAD - N), (0, 0)))

    ones_deg = jnp.ones((CH,), jnp.float32)
    zeros_deg = jnp.zeros((RPT,), jnp.float32)
    zeros32 = jnp.zeros((RPT, HID_DIM), jnp.float32)
    zeros16 = jnp.zeros((RPT, OUT_DIM), jnp.float32)

    degp = _degree_call()(dst, ones_deg, zeros_deg)
    h1 = _tc_m_call(x_pad, W1)
    m1, dinv = _tc_a_call(h1, degp)
    acc1 = _make_sc_agg(HID_DIM)(src, dst, m1, zeros32)
    m2 = _tc_b_call(m1, acc1, dinv, b1.reshape(1, HID_DIM), W2)
    acc2 = _make_sc_agg(OUT_DIM)(src, dst, m2, zeros16)
    return _tc_c_call(m2, acc2, dinv, b2.reshape(1, OUT_DIM))


# speedup vs baseline: 52.7176x; 1.0259x over previous
import functools

import jax
import jax.numpy as jnp
from jax import lax
from jax.experimental import pallas as pl
from jax.experimental.pallas import tpu as pltpu
from jax.experimental.pallas import tpu_sc as plsc

N = 10000
E = 320000
IN_DIM = 128
HID_DIM = 32
OUT_DIM = 16

NPAD = 10240
NC = 2
NS = 16
NW = NC * NS
CH = 125
NCH = 80
EPW = NCH * CH
RPT = NPAD // NS
LAST_RPT = N - (NS - 1) * RPT


@functools.lru_cache(maxsize=None)
def _mesh():
    return plsc.VectorSubcoreMesh(core_axis_name="c", subcore_axis_name="s")


def _sc_degree(ei_hbm, ones_hbm, zeros_hbm, out_hbm, acc_sh, dst_v, ones_v, sem):
    cid = lax.axis_index("c")
    sid = lax.axis_index("s")
    wid = cid * NS + sid
    r0 = sid * RPT
    pltpu.sync_copy(zeros_hbm, acc_sh.at[pl.ds(r0, RPT)])
    pltpu.sync_copy(ei_hbm.at[1, wid], dst_v)
    pltpu.sync_copy(ones_hbm, ones_v)
    plsc.subcore_barrier()

    def step(j, carry):
        pltpu.sync_copy(ones_v, acc_sh.at[dst_v.at[j]], add=True)
        return carry

    lax.fori_loop(0, NCH, step, 0)
    plsc.subcore_barrier()
    pltpu.sync_copy(acc_sh.at[pl.ds(r0, RPT)], out_hbm.at[cid, pl.ds(r0, RPT)])


@functools.lru_cache(maxsize=None)
def _degree_call():
    return functools.partial(
        pl.kernel,
        out_type=jax.ShapeDtypeStruct((NC, NPAD), jnp.float32),
        mesh=_mesh(),
        compiler_params=pltpu.CompilerParams(use_tc_tiling_on_sc=False),
        scratch_types=[
            pltpu.VMEM_SHARED((NPAD,), jnp.float32),
            pltpu.VMEM((NCH, CH), jnp.int32),
            pltpu.VMEM((CH,), jnp.float32),
            pltpu.SemaphoreType.DMA,
        ],
    )(_sc_degree)


def _make_sc_agg(D):
    def _sc_agg(ei_hbm, m_hbm, zeros_hbm, out_hbm,
                tab_sh, acc_sh, src_v, dst_v, buf_a, buf_b, sem_a, sem_b):
        cid = lax.axis_index("c")
        sid = lax.axis_index("s")
        wid = cid * NS + sid
        r0 = sid * RPT
        @pl.when(sid < NS - 1)
        def _():
            pltpu.sync_copy(m_hbm.at[pl.ds(r0, RPT)], tab_sh.at[pl.ds(r0, RPT)])

        @pl.when(sid == NS - 1)
        def _():
            pltpu.sync_copy(m_hbm.at[pl.ds(r0, LAST_RPT)],
                            tab_sh.at[pl.ds(r0, LAST_RPT)])

        pltpu.sync_copy(zeros_hbm, acc_sh.at[pl.ds(r0, RPT)])
        pltpu.sync_copy(ei_hbm.at[0, wid], src_v)
        pltpu.sync_copy(ei_hbm.at[1, wid], dst_v)
        plsc.subcore_barrier()

        pltpu.async_copy(tab_sh.at[src_v.at[0]], buf_a, sem_a)
        pltpu.async_copy(tab_sh.at[src_v.at[1]], buf_b, sem_b)

        def step(t, carry):
            j0 = 2 * t
            j1 = 2 * t + 1
            pltpu.make_async_copy(tab_sh.at[src_v.at[j0]], buf_a, sem_a).wait()
            pltpu.sync_copy(buf_a, acc_sh.at[dst_v.at[j0]], add=True)
            nxt_a = jnp.minimum(j0 + 2, NCH - 1)
            pltpu.async_copy(tab_sh.at[src_v.at[nxt_a]], buf_a, sem_a)
            pltpu.make_async_copy(tab_sh.at[src_v.at[j1]], buf_b, sem_b).wait()
            pltpu.sync_copy(buf_b, acc_sh.at[dst_v.at[j1]], add=True)
            nxt_b = jnp.minimum(j1 + 2, NCH - 1)
            pltpu.async_copy(tab_sh.at[src_v.at[nxt_b]], buf_b, sem_b)
            return carry

        lax.fori_loop(0, NCH // 2, step, 0)
        pltpu.make_async_copy(tab_sh.at[src_v.at[0]], buf_a, sem_a).wait()
        pltpu.make_async_copy(tab_sh.at[src_v.at[0]], buf_b, sem_b).wait()
        plsc.subcore_barrier()
        pltpu.sync_copy(acc_sh.at[pl.ds(r0, RPT)],
                        out_hbm.at[cid, pl.ds(r0, RPT)])

    return functools.partial(
        pl.kernel,
        out_type=jax.ShapeDtypeStruct((NC, NPAD, D), jnp.float32),
        mesh=_mesh(),
        compiler_params=pltpu.CompilerParams(use_tc_tiling_on_sc=False),
        scratch_types=[
            pltpu.VMEM_SHARED((NPAD, D), jnp.float32),
            pltpu.VMEM_SHARED((NPAD, D), jnp.float32),
            pltpu.VMEM((NCH, CH), jnp.int32),
            pltpu.VMEM((NCH, CH), jnp.int32),
            pltpu.VMEM((CH, D), jnp.float32),
            pltpu.VMEM((CH, D), jnp.float32),
            pltpu.SemaphoreType.DMA,
            pltpu.SemaphoreType.DMA,
        ],
    )(_sc_agg)


_make_sc_agg = functools.lru_cache(maxsize=None)(_make_sc_agg)


_BLK = 2000
_GRID = N // _BLK


def _tc_m(x_ref, w1_ref, h1_ref):
    h1_ref[...] = jnp.dot(x_ref[...], w1_ref[...],
                          preferred_element_type=jnp.float32)


def _tc_a(h1_ref, degp_ref, m1_ref, dinv_ref):
    deg = degp_ref[0] + degp_ref[1] + 1.0
    dinv = lax.rsqrt(deg)
    m1_ref[...] = dinv * h1_ref[...]
    dinv_ref[...] = dinv


def _tc_b(m1_ref, acc_ref, dinv_ref, b1_ref, w2_ref, m2_ref):
    dinv = dinv_ref[...]
    s = m1_ref[...] + acc_ref[0] + acc_ref[1]
    z = jnp.maximum(dinv * s + b1_ref[...], 0.0)
    m2_ref[...] = dinv * jnp.dot(z, w2_ref[...], preferred_element_type=jnp.float32)


def _tc_c(m2_ref, acc_ref, dinv_ref, b2_ref, out_ref):
    dinv = dinv_ref[...]
    s = m2_ref[...] + acc_ref[0] + acc_ref[1]
    out_ref[...] = dinv * s + b2_ref[...]


def _tc_m_call(x, W1):
    return pl.pallas_call(
        _tc_m,
        grid=(_GRID,),
        in_specs=[
            pl.BlockSpec((_BLK, IN_DIM), lambda i: (i, 0)),
            pl.BlockSpec((IN_DIM, HID_DIM), lambda i: (0, 0)),
        ],
        out_specs=pl.BlockSpec((_BLK, HID_DIM), lambda i: (i, 0)),
        out_shape=jax.ShapeDtypeStruct((N, HID_DIM), jnp.float32),
    )(x, W1)


def _tc_a_call(h1, degp):
    return pl.pallas_call(
        _tc_a,
        grid=(_GRID,),
        in_specs=[
            pl.BlockSpec((_BLK, HID_DIM), lambda i: (i, 0)),
            pl.BlockSpec((NC, _BLK, 1), lambda i: (0, i, 0)),
        ],
        out_specs=[
            pl.BlockSpec((_BLK, HID_DIM), lambda i: (i, 0)),
            pl.BlockSpec((_BLK, 1), lambda i: (i, 0)),
        ],
        out_shape=[
            jax.ShapeDtypeStruct((N, HID_DIM), jnp.float32),
            jax.ShapeDtypeStruct((N, 1), jnp.float32),
        ],
    )(h1, degp)


def _tc_b_call(m1, acc1, dinv, b1, W2):
    return pl.pallas_call(
        _tc_b,
        grid=(_GRID,),
        in_specs=[
            pl.BlockSpec((_BLK, HID_DIM), lambda i: (i, 0)),
            pl.BlockSpec((NC, _BLK, HID_DIM), lambda i: (0, i, 0)),
            pl.BlockSpec((_BLK, 1), lambda i: (i, 0)),
            pl.BlockSpec((1, HID_DIM), lambda i: (0, 0)),
            pl.BlockSpec((HID_DIM, OUT_DIM), lambda i: (0, 0)),
        ],
        out_specs=pl.BlockSpec((_BLK, OUT_DIM), lambda i: (i, 0)),
        out_shape=jax.ShapeDtypeStruct((N, OUT_DIM), jnp.float32),
    )(m1, acc1, dinv, b1, W2)


def _tc_c_call(m2, acc2, dinv, b2):
    return pl.pallas_call(
        _tc_c,
        grid=(_GRID,),
        in_specs=[
            pl.BlockSpec((_BLK, OUT_DIM), lambda i: (i, 0)),
            pl.BlockSpec((NC, _BLK, OUT_DIM), lambda i: (0, i, 0)),
            pl.BlockSpec((_BLK, 1), lambda i: (i, 0)),
            pl.BlockSpec((1, OUT_DIM), lambda i: (0, 0)),
        ],
        out_specs=pl.BlockSpec((_BLK, OUT_DIM), lambda i: (i, 0)),
        out_shape=jax.ShapeDtypeStruct((N, OUT_DIM), jnp.float32),
    )(m2, acc2, dinv, b2)


def kernel(x, edge_index, W1, b1, W2, b2):
    ei = edge_index.reshape(2, NW, NCH, CH)

    ones_deg = jnp.ones((CH,), jnp.float32)
    zeros_deg = jnp.zeros((RPT,), jnp.float32)
    zeros32 = jnp.zeros((RPT, HID_DIM), jnp.float32)
    zeros16 = jnp.zeros((RPT, OUT_DIM), jnp.float32)

    degp = _degree_call()(ei, ones_deg, zeros_deg)
    h1 = _tc_m_call(x, W1)
    m1, dinv = _tc_a_call(h1, degp.reshape(NC, NPAD, 1))
    acc1 = _make_sc_agg(HID_DIM)(ei, m1, zeros32)
    m2 = _tc_b_call(m1, acc1, dinv, b1.reshape(1, HID_DIM), W2)
    acc2 = _make_sc_agg(OUT_DIM)(ei, m2, zeros16)
    return _tc_c_call(m2, acc2, dinv, b2.reshape(1, OUT_DIM))


# speedup vs baseline: 56.5658x; 1.0730x over previous
import functools

import jax
import jax.numpy as jnp
from jax import lax
from jax.experimental import pallas as pl
from jax.experimental.pallas import tpu as pltpu
from jax.experimental.pallas import tpu_sc as plsc

N = 10000
E = 320000
IN_DIM = 128
HID_DIM = 32
OUT_DIM = 16

NPAD = 10240
NC = 2
NS = 16
NW = NC * NS
CH = 125
NCH = 80
EPW = NCH * CH
RPT = NPAD // NS
LAST_RPT = N - (NS - 1) * RPT


@functools.lru_cache(maxsize=None)
def _mesh():
    return plsc.VectorSubcoreMesh(core_axis_name="c", subcore_axis_name="s")


def _sc_degree(ei_hbm, ones_hbm, zeros_hbm, out_hbm, acc_sh, dst_v, ones_v, sem):
    cid = lax.axis_index("c")
    sid = lax.axis_index("s")
    wid = cid * NS + sid
    r0 = sid * RPT
    pltpu.sync_copy(zeros_hbm, acc_sh.at[pl.ds(r0, RPT)])
    pltpu.sync_copy(ei_hbm.at[1, wid], dst_v)
    pltpu.sync_copy(ones_hbm, ones_v)
    plsc.subcore_barrier()

    def step(j, carry):
        pltpu.sync_copy(ones_v, acc_sh.at[dst_v.at[j]], add=True)
        return carry

    lax.fori_loop(0, NCH, step, 0)
    plsc.subcore_barrier()
    pltpu.sync_copy(acc_sh.at[pl.ds(r0, RPT)], out_hbm.at[cid, pl.ds(r0, RPT)])


@functools.lru_cache(maxsize=None)
def _degree_call():
    return functools.partial(
        pl.kernel,
        out_type=jax.ShapeDtypeStruct((NC, NPAD), jnp.float32),
        mesh=_mesh(),
        compiler_params=pltpu.CompilerParams(use_tc_tiling_on_sc=False),
        scratch_types=[
            pltpu.VMEM_SHARED((NPAD,), jnp.float32),
            pltpu.VMEM((NCH, CH), jnp.int32),
            pltpu.VMEM((CH,), jnp.float32),
            pltpu.SemaphoreType.DMA,
        ],
    )(_sc_degree)


def _make_sc_agg(D):
    def _sc_agg(ei_hbm, m_hbm, zeros_hbm, out_hbm,
                tab_sh, acc_sh, src_v, dst_v, buf_a, buf_b, sem_a, sem_b):
        cid = lax.axis_index("c")
        sid = lax.axis_index("s")
        wid = cid * NS + sid
        r0 = sid * RPT
        @pl.when(sid < NS - 1)
        def _():
            pltpu.sync_copy(m_hbm.at[pl.ds(r0, RPT)], tab_sh.at[pl.ds(r0, RPT)])

        @pl.when(sid == NS - 1)
        def _():
            pltpu.sync_copy(m_hbm.at[pl.ds(r0, LAST_RPT)],
                            tab_sh.at[pl.ds(r0, LAST_RPT)])

        pltpu.sync_copy(zeros_hbm, acc_sh.at[pl.ds(r0, RPT)])
        pltpu.sync_copy(ei_hbm.at[0, wid], src_v)
        pltpu.sync_copy(ei_hbm.at[1, wid], dst_v)
        plsc.subcore_barrier()

        pltpu.async_copy(tab_sh.at[src_v.at[0]], buf_a, sem_a)
        pltpu.async_copy(tab_sh.at[src_v.at[1]], buf_b, sem_b)

        def step(t, carry):
            j0 = 2 * t
            j1 = 2 * t + 1
            pltpu.make_async_copy(tab_sh.at[src_v.at[j0]], buf_a, sem_a).wait()
            pltpu.sync_copy(buf_a, acc_sh.at[dst_v.at[j0]], add=True)
            nxt_a = jnp.minimum(j0 + 2, NCH - 1)
            pltpu.async_copy(tab_sh.at[src_v.at[nxt_a]], buf_a, sem_a)
            pltpu.make_async_copy(tab_sh.at[src_v.at[j1]], buf_b, sem_b).wait()
            pltpu.sync_copy(buf_b, acc_sh.at[dst_v.at[j1]], add=True)
            nxt_b = jnp.minimum(j1 + 2, NCH - 1)
            pltpu.async_copy(tab_sh.at[src_v.at[nxt_b]], buf_b, sem_b)
            return carry

        lax.fori_loop(0, NCH // 2, step, 0)
        pltpu.make_async_copy(tab_sh.at[src_v.at[0]], buf_a, sem_a).wait()
        pltpu.make_async_copy(tab_sh.at[src_v.at[0]], buf_b, sem_b).wait()
        plsc.subcore_barrier()
        pltpu.sync_copy(acc_sh.at[pl.ds(r0, RPT)],
                        out_hbm.at[cid, pl.ds(r0, RPT)])

    return functools.partial(
        pl.kernel,
        out_type=jax.ShapeDtypeStruct((NC, NPAD, D), jnp.float32),
        mesh=_mesh(),
        compiler_params=pltpu.CompilerParams(use_tc_tiling_on_sc=False),
        scratch_types=[
            pltpu.VMEM_SHARED((NPAD, D), jnp.float32),
            pltpu.VMEM_SHARED((NPAD, D), jnp.float32),
            pltpu.VMEM((NCH, CH), jnp.int32),
            pltpu.VMEM((NCH, CH), jnp.int32),
            pltpu.VMEM((CH, D), jnp.float32),
            pltpu.VMEM((CH, D), jnp.float32),
            pltpu.SemaphoreType.DMA,
            pltpu.SemaphoreType.DMA,
        ],
    )(_sc_agg)


_make_sc_agg = functools.lru_cache(maxsize=None)(_make_sc_agg)


_BLK = 2048
_GRID = 5


def _tc_m(x_ref, w1_ref, h1_ref):
    h1_ref[...] = jnp.dot(x_ref[...], w1_ref[...],
                          preferred_element_type=jnp.float32)


def _tc_a(h1_ref, degp_ref, m1_ref, dinv_ref):
    deg = degp_ref[0] + degp_ref[1] + 1.0
    dinv = lax.rsqrt(deg).reshape(_BLK, 1)
    m1_ref[...] = dinv * h1_ref[...]
    dinv_ref[...] = dinv


def _tc_b(m1_ref, acc_ref, dinv_ref, b1_ref, w2_ref, m2_ref):
    dinv = dinv_ref[...]
    s = m1_ref[...] + acc_ref[0] + acc_ref[1]
    z = jnp.maximum(dinv * s + b1_ref[...], 0.0)
    m2_ref[...] = dinv * jnp.dot(z, w2_ref[...], preferred_element_type=jnp.float32)


def _tc_c(m2_ref, acc_ref, dinv_ref, b2_ref, out_ref):
    dinv = dinv_ref[...]
    s = m2_ref[...] + acc_ref[0] + acc_ref[1]
    out_ref[...] = dinv * s + b2_ref[...]


def _tc_m_call(x, W1):
    return pl.pallas_call(
        _tc_m,
        grid=(_GRID,),
        in_specs=[
            pl.BlockSpec((_BLK, IN_DIM), lambda i: (i, 0)),
            pl.BlockSpec((IN_DIM, HID_DIM), lambda i: (0, 0)),
        ],
        out_specs=pl.BlockSpec((_BLK, HID_DIM), lambda i: (i, 0)),
        out_shape=jax.ShapeDtypeStruct((N, HID_DIM), jnp.float32),
    )(x, W1)


def _tc_a_call(h1, degp):
    return pl.pallas_call(
        _tc_a,
        grid=(_GRID,),
        in_specs=[
            pl.BlockSpec((_BLK, HID_DIM), lambda i: (i, 0)),
            pl.BlockSpec((NC, _BLK), lambda i: (0, i)),
        ],
        out_specs=[
            pl.BlockSpec((_BLK, HID_DIM), lambda i: (i, 0)),
            pl.BlockSpec((_BLK, 1), lambda i: (i, 0)),
        ],
        out_shape=[
            jax.ShapeDtypeStruct((N, HID_DIM), jnp.float32),
            jax.ShapeDtypeStruct((N, 1), jnp.float32),
        ],
    )(h1, degp)


def _tc_b_call(m1, acc1, dinv, b1, W2):
    return pl.pallas_call(
        _tc_b,
        grid=(_GRID,),
        in_specs=[
            pl.BlockSpec((_BLK, HID_DIM), lambda i: (i, 0)),
            pl.BlockSpec((NC, _BLK, HID_DIM), lambda i: (0, i, 0)),
            pl.BlockSpec((_BLK, 1), lambda i: (i, 0)),
            pl.BlockSpec((1, HID_DIM), lambda i: (0, 0)),
            pl.BlockSpec((HID_DIM, OUT_DIM), lambda i: (0, 0)),
        ],
        out_specs=pl.BlockSpec((_BLK, OUT_DIM), lambda i: (i, 0)),
        out_shape=jax.ShapeDtypeStruct((N, OUT_DIM), jnp.float32),
    )(m1, acc1, dinv, b1, W2)


def _tc_c_call(m2, acc2, dinv, b2):
    return pl.pallas_call(
        _tc_c,
        grid=(_GRID,),
        in_specs=[
            pl.BlockSpec((_BLK, OUT_DIM), lambda i: (i, 0)),
            pl.BlockSpec((NC, _BLK, OUT_DIM), lambda i: (0, i, 0)),
            pl.BlockSpec((_BLK, 1), lambda i: (i, 0)),
            pl.BlockSpec((1, OUT_DIM), lambda i: (0, 0)),
        ],
        out_specs=pl.BlockSpec((_BLK, OUT_DIM), lambda i: (i, 0)),
        out_shape=jax.ShapeDtypeStruct((N, OUT_DIM), jnp.float32),
    )(m2, acc2, dinv, b2)


def kernel(x, edge_index, W1, b1, W2, b2):
    ei = edge_index.reshape(2, NW, NCH, CH)

    ones_deg = jnp.ones((CH,), jnp.float32)
    zeros_deg = jnp.zeros((RPT,), jnp.float32)
    zeros32 = jnp.zeros((RPT, HID_DIM), jnp.float32)
    zeros16 = jnp.zeros((RPT, OUT_DIM), jnp.float32)

    degp = _degree_call()(ei, ones_deg, zeros_deg)
    h1 = _tc_m_call(x, W1)
    m1, dinv = _tc_a_call(h1, degp)
    acc1 = _make_sc_agg(HID_DIM)(ei, m1, zeros32)
    m2 = _tc_b_call(m1, acc1, dinv, b1.reshape(1, HID_DIM), W2)
    acc2 = _make_sc_agg(OUT_DIM)(ei, m2, zeros16)
    return _tc_c_call(m2, acc2, dinv, b2.reshape(1, OUT_DIM))


# speedup vs baseline: 56.8704x; 1.0054x over previous
import functools

import jax
import jax.numpy as jnp
from jax import lax
from jax.experimental import pallas as pl
from jax.experimental.pallas import tpu as pltpu
from jax.experimental.pallas import tpu_sc as plsc

N = 10000
E = 320000
IN_DIM = 128
HID_DIM = 32
OUT_DIM = 16

NPAD = 10240
NC = 2
NS = 16
NW = NC * NS
CH = 125
NCH = 80
EPW = NCH * CH
RPT = NPAD // NS
LAST_RPT = N - (NS - 1) * RPT


@functools.lru_cache(maxsize=None)
def _mesh():
    return plsc.VectorSubcoreMesh(core_axis_name="c", subcore_axis_name="s")


def _sc_degree(ei_hbm, ones_hbm, zeros_hbm, out_hbm, acc_sh, dst_v, ones_v, sem):
    cid = lax.axis_index("c")
    sid = lax.axis_index("s")
    wid = cid * NS + sid
    r0 = sid * RPT
    pltpu.sync_copy(zeros_hbm, acc_sh.at[pl.ds(r0, RPT)])
    pltpu.sync_copy(ei_hbm.at[1, wid], dst_v)
    pltpu.sync_copy(ones_hbm, ones_v)
    plsc.subcore_barrier()

    def step(j, carry):
        pltpu.sync_copy(ones_v, acc_sh.at[dst_v.at[j]], add=True)
        return carry

    lax.fori_loop(0, NCH, step, 0)
    plsc.subcore_barrier()
    pltpu.sync_copy(acc_sh.at[pl.ds(r0, RPT)], out_hbm.at[cid, pl.ds(r0, RPT)])


@functools.lru_cache(maxsize=None)
def _degree_call():
    return functools.partial(
        pl.kernel,
        out_type=jax.ShapeDtypeStruct((NC, NPAD), jnp.float32),
        mesh=_mesh(),
        compiler_params=pltpu.CompilerParams(use_tc_tiling_on_sc=False),
        scratch_types=[
            pltpu.VMEM_SHARED((NPAD,), jnp.float32),
            pltpu.VMEM((NCH, CH), jnp.int32),
            pltpu.VMEM((CH,), jnp.float32),
            pltpu.SemaphoreType.DMA,
        ],
    )(_sc_degree)


def _make_sc_agg(D):
    def _sc_agg(ei_hbm, m_hbm, zeros_hbm, out_hbm,
                tab_sh, acc_sh, src_v, dst_v,
                buf_a, buf_b, buf_c, buf_d, sem_a, sem_b, sem_c, sem_d):
        cid = lax.axis_index("c")
        sid = lax.axis_index("s")
        wid = cid * NS + sid
        r0 = sid * RPT
        @pl.when(sid < NS - 1)
        def _():
            pltpu.sync_copy(m_hbm.at[pl.ds(r0, RPT)], tab_sh.at[pl.ds(r0, RPT)])

        @pl.when(sid == NS - 1)
        def _():
            pltpu.sync_copy(m_hbm.at[pl.ds(r0, LAST_RPT)],
                            tab_sh.at[pl.ds(r0, LAST_RPT)])

        pltpu.sync_copy(zeros_hbm, acc_sh.at[pl.ds(r0, RPT)])
        pltpu.sync_copy(ei_hbm.at[0, wid], src_v)
        pltpu.sync_copy(ei_hbm.at[1, wid], dst_v)
        plsc.subcore_barrier()

        bufs = (buf_a, buf_b, buf_c, buf_d)
        sems = (sem_a, sem_b, sem_c, sem_d)
        for k in range(4):
            pltpu.async_copy(tab_sh.at[src_v.at[k]], bufs[k], sems[k])

        def step(t, carry):
            for k in range(4):
                j = 4 * t + k
                pltpu.make_async_copy(tab_sh.at[src_v.at[j]],
                                      bufs[k], sems[k]).wait()
                pltpu.sync_copy(bufs[k], acc_sh.at[dst_v.at[j]], add=True)
                nxt = jnp.minimum(j + 4, NCH - 1)
                pltpu.async_copy(tab_sh.at[src_v.at[nxt]], bufs[k], sems[k])
            return carry

        lax.fori_loop(0, NCH // 4, step, 0)
        for k in range(4):
            pltpu.make_async_copy(tab_sh.at[src_v.at[0]], bufs[k], sems[k]).wait()
        plsc.subcore_barrier()
        pltpu.sync_copy(acc_sh.at[pl.ds(r0, RPT)],
                        out_hbm.at[cid, pl.ds(r0, RPT)])

    return functools.partial(
        pl.kernel,
        out_type=jax.ShapeDtypeStruct((NC, NPAD, D), jnp.float32),
        mesh=_mesh(),
        compiler_params=pltpu.CompilerParams(use_tc_tiling_on_sc=False),
        scratch_types=[
            pltpu.VMEM_SHARED((NPAD, D), jnp.float32),
            pltpu.VMEM_SHARED((NPAD, D), jnp.float32),
            pltpu.VMEM((NCH, CH), jnp.int32),
            pltpu.VMEM((NCH, CH), jnp.int32),
            pltpu.VMEM((CH, D), jnp.float32),
            pltpu.VMEM((CH, D), jnp.float32),
            pltpu.VMEM((CH, D), jnp.float32),
            pltpu.VMEM((CH, D), jnp.float32),
            pltpu.SemaphoreType.DMA,
            pltpu.SemaphoreType.DMA,
            pltpu.SemaphoreType.DMA,
            pltpu.SemaphoreType.DMA,
        ],
    )(_sc_agg)


_make_sc_agg = functools.lru_cache(maxsize=None)(_make_sc_agg)


_BLK = 2048
_GRID = 5


def _tc_m(x_ref, w1_ref, h1_ref):
    h1_ref[...] = jnp.dot(x_ref[...], w1_ref[...],
                          preferred_element_type=jnp.float32)


def _tc_a(h1_ref, degp_ref, m1_ref, dinv_ref):
    deg = degp_ref[0] + degp_ref[1] + 1.0
    dinv = lax.rsqrt(deg).reshape(_BLK, 1)
    m1_ref[...] = dinv * h1_ref[...]
    dinv_ref[...] = dinv


def _tc_b(m1_ref, acc_ref, dinv_ref, b1_ref, w2_ref, m2_ref):
    dinv = dinv_ref[...]
    s = m1_ref[...] + acc_ref[0] + acc_ref[1]
    z = jnp.maximum(dinv * s + b1_ref[...], 0.0)
    m2_ref[...] = dinv * jnp.dot(z, w2_ref[...], preferred_element_type=jnp.float32)


def _tc_c(m2_ref, acc_ref, dinv_ref, b2_ref, out_ref):
    dinv = dinv_ref[...]
    s = m2_ref[...] + acc_ref[0] + acc_ref[1]
    out_ref[...] = dinv * s + b2_ref[...]


def _tc_m_call(x, W1):
    return pl.pallas_call(
        _tc_m,
        grid=(_GRID,),
        in_specs=[
            pl.BlockSpec((_BLK, IN_DIM), lambda i: (i, 0)),
            pl.BlockSpec((IN_DIM, HID_DIM), lambda i: (0, 0)),
        ],
        out_specs=pl.BlockSpec((_BLK, HID_DIM), lambda i: (i, 0)),
        out_shape=jax.ShapeDtypeStruct((N, HID_DIM), jnp.float32),
    )(x, W1)


def _tc_a_call(h1, degp):
    return pl.pallas_call(
        _tc_a,
        grid=(_GRID,),
        in_specs=[
            pl.BlockSpec((_BLK, HID_DIM), lambda i: (i, 0)),
            pl.BlockSpec((NC, _BLK), lambda i: (0, i)),
        ],
        out_specs=[
            pl.BlockSpec((_BLK, HID_DIM), lambda i: (i, 0)),
            pl.BlockSpec((_BLK, 1), lambda i: (i, 0)),
        ],
        out_shape=[
            jax.ShapeDtypeStruct((N, HID_DIM), jnp.float32),
            jax.ShapeDtypeStruct((N, 1), jnp.float32),
        ],
    )(h1, degp)


def _tc_b_call(m1, acc1, dinv, b1, W2):
    return pl.pallas_call(
        _tc_b,
        grid=(_GRID,),
        in_specs=[
            pl.BlockSpec((_BLK, HID_DIM), lambda i: (i, 0)),
            pl.BlockSpec((NC, _BLK, HID_DIM), lambda i: (0, i, 0)),
            pl.BlockSpec((_BLK, 1), lambda i: (i, 0)),
            pl.BlockSpec((1, HID_DIM), lambda i: (0, 0)),
            pl.BlockSpec((HID_DIM, OUT_DIM), lambda i: (0, 0)),
        ],
        out_specs=pl.BlockSpec((_BLK, OUT_DIM), lambda i: (i, 0)),
        out_shape=jax.ShapeDtypeStruct((N, OUT_DIM), jnp.float32),
    )(m1, acc1, dinv, b1, W2)


def _tc_c_call(m2, acc2, dinv, b2):
    return pl.pallas_call(
        _tc_c,
        grid=(_GRID,),
        in_specs=[
            pl.BlockSpec((_BLK, OUT_DIM), lambda i: (i, 0)),
            pl.BlockSpec((NC, _BLK, OUT_DIM), lambda i: (0, i, 0)),
            pl.BlockSpec((_BLK, 1), lambda i: (i, 0)),
            pl.BlockSpec((1, OUT_DIM), lambda i: (0, 0)),
        ],
        out_specs=pl.BlockSpec((_BLK, OUT_DIM), lambda i: (i, 0)),
        out_shape=jax.ShapeDtypeStruct((N, OUT_DIM), jnp.float32),
    )(m2, acc2, dinv, b2)


def kernel(x, edge_index, W1, b1, W2, b2):
    ei = edge_index.reshape(2, NW, NCH, CH)

    ones_deg = jnp.ones((CH,), jnp.float32)
    zeros_deg = jnp.zeros((RPT,), jnp.float32)
    zeros32 = jnp.zeros((RPT, HID_DIM), jnp.float32)
    zeros16 = jnp.zeros((RPT, OUT_DIM), jnp.float32)

    degp = _degree_call()(ei, ones_deg, zeros_deg)
    h1 = _tc_m_call(x, W1)
    m1, dinv = _tc_a_call(h1, degp)
    acc1 = _make_sc_agg(HID_DIM)(ei, m1, zeros32)
    m2 = _tc_b_call(m1, acc1, dinv, b1.reshape(1, HID_DIM), W2)
    acc2 = _make_sc_agg(OUT_DIM)(ei, m2, zeros16)
    return _tc_c_call(m2, acc2, dinv, b2.reshape(1, OUT_DIM))


# speedup vs baseline: 57.3339x; 1.0082x over previous
import functools

import jax
import jax.numpy as jnp
from jax import lax
from jax.experimental import pallas as pl
from jax.experimental.pallas import tpu as pltpu
from jax.experimental.pallas import tpu_sc as plsc

N = 10000
E = 320000
IN_DIM = 128
HID_DIM = 32
OUT_DIM = 16

NPAD = 10240
NC = 2
NS = 16
NW = NC * NS
CH = 125
NCH = 80
EPW = NCH * CH
RPT = NPAD // NS
LAST_RPT = N - (NS - 1) * RPT


@functools.lru_cache(maxsize=None)
def _mesh():
    return plsc.VectorSubcoreMesh(core_axis_name="c", subcore_axis_name="s")


def _sc_degree(ei_hbm, ones_hbm, zeros_hbm, out_hbm, acc_sh, dst_v, ones_v, sem):
    cid = lax.axis_index("c")
    sid = lax.axis_index("s")
    wid = cid * NS + sid
    r0 = sid * RPT
    pltpu.sync_copy(zeros_hbm, acc_sh.at[pl.ds(r0, RPT)])
    pltpu.sync_copy(ei_hbm.at[1, wid], dst_v)
    pltpu.sync_copy(ones_hbm, ones_v)
    plsc.subcore_barrier()

    def step(j, carry):
        pltpu.sync_copy(ones_v, acc_sh.at[dst_v.at[j]], add=True)
        return carry

    lax.fori_loop(0, NCH, step, 0)
    plsc.subcore_barrier()

    @pl.when(sid < NS - 1)
    def _():
        pltpu.sync_copy(acc_sh.at[pl.ds(r0, RPT)],
                        out_hbm.at[cid, pl.ds(r0, RPT)])

    @pl.when(sid == NS - 1)
    def _():
        pltpu.sync_copy(acc_sh.at[pl.ds(r0, LAST_RPT)],
                        out_hbm.at[cid, pl.ds(r0, LAST_RPT)])


@functools.lru_cache(maxsize=None)
def _degree_call():
    return functools.partial(
        pl.kernel,
        out_type=jax.ShapeDtypeStruct((NC, N), jnp.float32),
        mesh=_mesh(),
        compiler_params=pltpu.CompilerParams(use_tc_tiling_on_sc=False),
        scratch_types=[
            pltpu.VMEM_SHARED((NPAD,), jnp.float32),
            pltpu.VMEM((NCH, CH), jnp.int32),
            pltpu.VMEM((CH,), jnp.float32),
            pltpu.SemaphoreType.DMA,
        ],
    )(_sc_degree)


def _make_sc_agg(D):
    def _sc_agg(ei_hbm, m_hbm, zeros_hbm, out_hbm,
                tab_sh, acc_sh, src_v, dst_v,
                buf_a, buf_b, buf_c, buf_d, sem_a, sem_b, sem_c, sem_d):
        cid = lax.axis_index("c")
        sid = lax.axis_index("s")
        wid = cid * NS + sid
        r0 = sid * RPT
        @pl.when(sid < NS - 1)
        def _():
            pltpu.sync_copy(m_hbm.at[pl.ds(r0, RPT)], tab_sh.at[pl.ds(r0, RPT)])

        @pl.when(sid == NS - 1)
        def _():
            pltpu.sync_copy(m_hbm.at[pl.ds(r0, LAST_RPT)],
                            tab_sh.at[pl.ds(r0, LAST_RPT)])

        pltpu.sync_copy(zeros_hbm, acc_sh.at[pl.ds(r0, RPT)])
        pltpu.sync_copy(ei_hbm.at[0, wid], src_v)
        pltpu.sync_copy(ei_hbm.at[1, wid], dst_v)
        plsc.subcore_barrier()

        bufs = (buf_a, buf_b, buf_c, buf_d)
        sems = (sem_a, sem_b, sem_c, sem_d)
        for k in range(4):
            pltpu.async_copy(tab_sh.at[src_v.at[k]], bufs[k], sems[k])

        def step(t, carry):
            for k in range(4):
                j = 4 * t + k
                pltpu.make_async_copy(tab_sh.at[src_v.at[j]],
                                      bufs[k], sems[k]).wait()
                pltpu.sync_copy(bufs[k], acc_sh.at[dst_v.at[j]], add=True)
                nxt = jnp.minimum(j + 4, NCH - 1)
                pltpu.async_copy(tab_sh.at[src_v.at[nxt]], bufs[k], sems[k])
            return carry

        lax.fori_loop(0, NCH // 4, step, 0)
        for k in range(4):
            pltpu.make_async_copy(tab_sh.at[src_v.at[0]], bufs[k], sems[k]).wait()
        plsc.subcore_barrier()
        pltpu.sync_copy(acc_sh.at[pl.ds(r0, RPT)],
                        out_hbm.at[cid, pl.ds(r0, RPT)])

    return functools.partial(
        pl.kernel,
        out_type=jax.ShapeDtypeStruct((NC, NPAD, D), jnp.float32),
        mesh=_mesh(),
        compiler_params=pltpu.CompilerParams(use_tc_tiling_on_sc=False),
        scratch_types=[
            pltpu.VMEM_SHARED((NPAD, D), jnp.float32),
            pltpu.VMEM_SHARED((NPAD, D), jnp.float32),
            pltpu.VMEM((NCH, CH), jnp.int32),
            pltpu.VMEM((NCH, CH), jnp.int32),
            pltpu.VMEM((CH, D), jnp.float32),
            pltpu.VMEM((CH, D), jnp.float32),
            pltpu.VMEM((CH, D), jnp.float32),
            pltpu.VMEM((CH, D), jnp.float32),
            pltpu.SemaphoreType.DMA,
            pltpu.SemaphoreType.DMA,
            pltpu.SemaphoreType.DMA,
            pltpu.SemaphoreType.DMA,
        ],
    )(_sc_agg)


_make_sc_agg = functools.lru_cache(maxsize=None)(_make_sc_agg)


_BLK = N
_GRID = 1


def _tc_m(x_ref, w1_ref, h1_ref):
    h1_ref[...] = jnp.dot(x_ref[...], w1_ref[...],
                          preferred_element_type=jnp.float32)


def _tc_a(h1_ref, degp_ref, m1_ref, dinv_ref):
    deg = degp_ref[0] + degp_ref[1] + 1.0
    dinv = lax.rsqrt(deg).reshape(_BLK, 1)
    m1_ref[...] = dinv * h1_ref[...]
    dinv_ref[...] = dinv


def _tc_b(m1_ref, acc_ref, dinv_ref, b1_ref, w2_ref, m2_ref):
    dinv = dinv_ref[...]
    s = m1_ref[...] + acc_ref[0] + acc_ref[1]
    z = jnp.maximum(dinv * s + b1_ref[...], 0.0)
    m2_ref[...] = dinv * jnp.dot(z, w2_ref[...], preferred_element_type=jnp.float32)


def _tc_c(m2_ref, acc_ref, dinv_ref, b2_ref, out_ref):
    dinv = dinv_ref[...]
    s = m2_ref[...] + acc_ref[0] + acc_ref[1]
    out_ref[...] = dinv * s + b2_ref[...]


def _tc_m_call(x, W1):
    return pl.pallas_call(
        _tc_m,
        grid=(_GRID,),
        in_specs=[
            pl.BlockSpec((_BLK, IN_DIM), lambda i: (i, 0)),
            pl.BlockSpec((IN_DIM, HID_DIM), lambda i: (0, 0)),
        ],
        out_specs=pl.BlockSpec((_BLK, HID_DIM), lambda i: (i, 0)),
        out_shape=jax.ShapeDtypeStruct((N, HID_DIM), jnp.float32),
    )(x, W1)


def _tc_a_call(h1, degp):
    return pl.pallas_call(
        _tc_a,
        grid=(_GRID,),
        in_specs=[
            pl.BlockSpec((_BLK, HID_DIM), lambda i: (i, 0)),
            pl.BlockSpec((NC, N), lambda i: (0, 0)),
        ],
        out_specs=[
            pl.BlockSpec((_BLK, HID_DIM), lambda i: (i, 0)),
            pl.BlockSpec((_BLK, 1), lambda i: (i, 0)),
        ],
        out_shape=[
            jax.ShapeDtypeStruct((N, HID_DIM), jnp.float32),
            jax.ShapeDtypeStruct((N, 1), jnp.float32),
        ],
    )(h1, degp)


def _tc_b_call(m1, acc1, dinv, b1, W2):
    return pl.pallas_call(
        _tc_b,
        grid=(_GRID,),
        in_specs=[
            pl.BlockSpec((_BLK, HID_DIM), lambda i: (i, 0)),
            pl.BlockSpec((NC, _BLK, HID_DIM), lambda i: (0, i, 0)),
            pl.BlockSpec((_BLK, 1), lambda i: (i, 0)),
            pl.BlockSpec((1, HID_DIM), lambda i: (0, 0)),
            pl.BlockSpec((HID_DIM, OUT_DIM), lambda i: (0, 0)),
        ],
        out_specs=pl.BlockSpec((_BLK, OUT_DIM), lambda i: (i, 0)),
        out_shape=jax.ShapeDtypeStruct((N, OUT_DIM), jnp.float32),
    )(m1, acc1, dinv, b1, W2)


def _tc_c_call(m2, acc2, dinv, b2):
    return pl.pallas_call(
        _tc_c,
        grid=(_GRID,),
        in_specs=[
            pl.BlockSpec((_BLK, OUT_DIM), lambda i: (i, 0)),
            pl.BlockSpec((NC, _BLK, OUT_DIM), lambda i: (0, i, 0)),
            pl.BlockSpec((_BLK, 1), lambda i: (i, 0)),
            pl.BlockSpec((1, OUT_DIM), lambda i: (0, 0)),
        ],
        out_specs=pl.BlockSpec((_BLK, OUT_DIM), lambda i: (i, 0)),
        out_shape=jax.ShapeDtypeStruct((N, OUT_DIM), jnp.float32),
    )(m2, acc2, dinv, b2)


def kernel(x, edge_index, W1, b1, W2, b2):
    ei = edge_index.reshape(2, NW, NCH, CH)

    ones_deg = jnp.ones((CH,), jnp.float32)
    zeros_deg = jnp.zeros((RPT,), jnp.float32)
    zeros32 = jnp.zeros((RPT, HID_DIM), jnp.float32)
    zeros16 = jnp.zeros((RPT, OUT_DIM), jnp.float32)

    degp = _degree_call()(ei, ones_deg, zeros_deg)
    h1 = _tc_m_call(x, W1)
    m1, dinv = _tc_a_call(h1, degp)
    acc1 = _make_sc_agg(HID_DIM)(ei, m1, zeros32)
    m2 = _tc_b_call(m1, acc1, dinv, b1.reshape(1, HID_DIM), W2)
    acc2 = _make_sc_agg(OUT_DIM)(ei, m2, zeros16)
    return _tc_c_call(m2, acc2, dinv, b2.reshape(1, OUT_DIM))
